# Initial kernel scaffold; baseline (speedup 1.0000x reference)
import jax
import jax.numpy as jnp
from jax.experimental import pallas as pl


def kernel(z, embedding_weight):
    raise NotImplementedError("write your pallas kernel here")


# speedup vs baseline: 1.4269x; 1.4269x over previous
import jax
import jax.numpy as jnp
from jax.experimental import pallas as pl
from jax.experimental.pallas import tpu as pltpu

_K = 1024
_C = 256
_B = 8
_P = 1024
_BETA = 0.25
_EPS = 1e-12
_ENT_EPS = 1e-4


def _vq_body(z_ref, ew_ref, zq_ref, idx_ref, loss_ref, ent_ref,
             ewn_ref, ewt_ref, usage_ref):
    b = pl.program_id(0)
    nb = pl.num_programs(0)

    @pl.when(b == 0)
    def _init():
        ew = ew_ref[...]
        norm = jnp.sqrt(jnp.sum(ew * ew, axis=1, keepdims=True))
        ewn = ew / jnp.maximum(norm, _EPS)
        ewn_ref[...] = ewn
        ewt_ref[...] = ewn.T
        usage_ref[...] = jnp.zeros_like(usage_ref)
        loss_ref[...] = jnp.zeros_like(loss_ref)

    z = z_ref[0]
    s2 = jnp.sum(z * z, axis=0, keepdims=True)
    zn = z / jnp.maximum(jnp.sqrt(s2), _EPS)
    ewn = ewn_ref[...]
    sim = jax.lax.dot_general(
        ewn.astype(jnp.bfloat16), zn.astype(jnp.bfloat16),
        (((1,), (0,)), ((), ())),
        preferred_element_type=jnp.float32)
    smax = jnp.max(sim, axis=0, keepdims=True)
    kiota = jax.lax.broadcasted_iota(jnp.int32, sim.shape, 0)
    idx = jnp.min(jnp.where(sim == smax, kiota, jnp.int32(2**30)),
                  axis=0, keepdims=True)
    idx_ref[0] = idx

    onehot = (kiota == idx).astype(jnp.float32)
    usage_ref[...] += jnp.sum(onehot, axis=1, keepdims=True)
    zq = jax.lax.dot_general(
        ewt_ref[...], onehot, (((1,), (0,)), ((), ())),
        preferred_element_type=jnp.float32,
        precision=jax.lax.Precision.HIGHEST)
    zq_ref[0] = zq
    diff = zq - zn
    loss_ref[...] += jnp.sum(diff * diff).reshape(1, 1)

    @pl.when(b == nb - 1)
    def _finish():
        total = jnp.float32(_B * _P)
        loss_ref[...] = (1.0 + _BETA) * (loss_ref[...] / total)
        pe = usage_ref[...] + _ENT_EPS
        probs = pe / jnp.sum(pe)
        ent_ref[...] = -jnp.sum(probs * jnp.log(probs)).reshape(1, 1)


def kernel(z, embedding_weight):
    zr = z.reshape(_B, _C, _P)
    zq, idx, loss, ent = pl.pallas_call(
        _vq_body,
        grid=(_B,),
        in_specs=[
            pl.BlockSpec((1, _C, _P), lambda b: (b, 0, 0)),
            pl.BlockSpec((_K, _C), lambda b: (0, 0)),
        ],
        out_specs=[
            pl.BlockSpec((1, _C, _P), lambda b: (b, 0, 0)),
            pl.BlockSpec((1, 1, _P), lambda b: (b, 0, 0)),
            pl.BlockSpec((1, 1), lambda b: (0, 0)),
            pl.BlockSpec((1, 1), lambda b: (0, 0)),
        ],
        out_shape=[
            jax.ShapeDtypeStruct((_B, _C, _P), jnp.float32),
            jax.ShapeDtypeStruct((_B, 1, _P), jnp.int32),
            jax.ShapeDtypeStruct((1, 1), jnp.float32),
            jax.ShapeDtypeStruct((1, 1), jnp.float32),
        ],
        scratch_shapes=[
            pltpu.VMEM((_K, _C), jnp.float32),
            pltpu.VMEM((_C, _K), jnp.float32),
            pltpu.VMEM((_K, 1), jnp.float32),
        ],
    )(zr, embedding_weight)
    return (zq.reshape(_B, _C, 32, 32), loss[0, 0], ent[0, 0],
            idx.reshape(_B, _P))


# speedup vs baseline: 1.9559x; 1.3707x over previous
import jax
import jax.numpy as jnp
from jax.experimental import pallas as pl
from jax.experimental.pallas import tpu as pltpu

_K = 1024
_C = 256
_B = 8
_P = 1024
_BETA = 0.25
_EPS = 1e-12
_ENT_EPS = 1e-4


def _vq_body(z_ref, ew_ref, zq_ref, idx_ref, loss_ref, ent_ref,
             ewn_ref, ewthi_ref, ewtlo_ref, usage_ref):
    b = pl.program_id(0)
    nb = pl.num_programs(0)

    @pl.when(b == 0)
    def _init():
        ew = ew_ref[...]
        norm = jnp.sqrt(jnp.sum(ew * ew, axis=1, keepdims=True))
        ewn = ew / jnp.maximum(norm, _EPS)
        ewn_ref[...] = ewn
        ewt = ewn.T
        hi = ewt.astype(jnp.bfloat16)
        ewthi_ref[...] = hi
        ewtlo_ref[...] = (ewt - hi.astype(jnp.float32)).astype(jnp.bfloat16)
        usage_ref[...] = jnp.zeros_like(usage_ref)
        loss_ref[...] = jnp.zeros_like(loss_ref)

    z = z_ref[0]
    s2 = jnp.sum(z * z, axis=0, keepdims=True)
    zn = z / jnp.maximum(jnp.sqrt(s2), _EPS)
    ewn = ewn_ref[...]
    sim = jax.lax.dot_general(
        ewn.astype(jnp.bfloat16), zn.astype(jnp.bfloat16),
        (((1,), (0,)), ((), ())),
        preferred_element_type=jnp.float32)
    smax = jnp.max(sim, axis=0, keepdims=True)
    kiota = jax.lax.broadcasted_iota(jnp.int32, sim.shape, 0)
    idx = jnp.min(jnp.where(sim == smax, kiota, jnp.int32(2**30)),
                  axis=0, keepdims=True)
    idx_ref[0] = idx

    onehot = (kiota == idx).astype(jnp.float32)
    usage_ref[...] += jnp.sum(onehot, axis=1, keepdims=True)
    oh16 = onehot.astype(jnp.bfloat16)
    cdims = (((1,), (0,)), ((), ()))
    zq = (jax.lax.dot_general(ewthi_ref[...], oh16, cdims,
                              preferred_element_type=jnp.float32)
          + jax.lax.dot_general(ewtlo_ref[...], oh16, cdims,
                                preferred_element_type=jnp.float32))
    zq_ref[0] = zq
    diff = zq - zn
    loss_ref[...] += jnp.sum(diff * diff).reshape(1, 1)

    @pl.when(b == nb - 1)
    def _finish():
        total = jnp.float32(_B * _P)
        loss_ref[...] = (1.0 + _BETA) * (loss_ref[...] / total)
        pe = usage_ref[...] + _ENT_EPS
        probs = pe / jnp.sum(pe)
        ent_ref[...] = -jnp.sum(probs * jnp.log(probs)).reshape(1, 1)


def kernel(z, embedding_weight):
    zr = z.reshape(_B, _C, _P)
    zq, idx, loss, ent = pl.pallas_call(
        _vq_body,
        grid=(_B,),
        in_specs=[
            pl.BlockSpec((1, _C, _P), lambda b: (b, 0, 0)),
            pl.BlockSpec((_K, _C), lambda b: (0, 0)),
        ],
        out_specs=[
            pl.BlockSpec((1, _C, _P), lambda b: (b, 0, 0)),
            pl.BlockSpec((1, 1, _P), lambda b: (b, 0, 0)),
            pl.BlockSpec((1, 1), lambda b: (0, 0)),
            pl.BlockSpec((1, 1), lambda b: (0, 0)),
        ],
        out_shape=[
            jax.ShapeDtypeStruct((_B, _C, _P), jnp.float32),
            jax.ShapeDtypeStruct((_B, 1, _P), jnp.int32),
            jax.ShapeDtypeStruct((1, 1), jnp.float32),
            jax.ShapeDtypeStruct((1, 1), jnp.float32),
        ],
        scratch_shapes=[
            pltpu.VMEM((_K, _C), jnp.float32),
            pltpu.VMEM((_C, _K), jnp.bfloat16),
            pltpu.VMEM((_C, _K), jnp.bfloat16),
            pltpu.VMEM((_K, 1), jnp.float32),
        ],
    )(zr, embedding_weight)
    return (zq.reshape(_B, _C, 32, 32), loss[0, 0], ent[0, 0],
            idx.reshape(_B, _P))
